# baseline (speedup 1.0000x reference)
import functools

import jax
import jax.numpy as jnp
from jax import lax
from jax.experimental import pallas as pl
from jax.experimental.pallas import tpu as pltpu
from jax.experimental.pallas import tpu_sc as plsc

N = 10000
E = 320000
D = 128

NC = 2
NS = 16
NW = NC * NS

CH = 128
HCH = CH // 2
CPW = 80
BS = 16
NB = CPW // BS
EPAD = NW * CPW * CH
ROWS_MAIN = 624
ROWS_LAST = 640
WCH = 48


def _logmap0_body(x_ref, o_ref):
    x = x_ref[...]
    n2 = jnp.sum(x * x, axis=1, keepdims=True)
    norm = jnp.maximum(jnp.sqrt(n2), 1e-15)
    t = jnp.clip(norm, -1.0 + 1e-7, 1.0 - 1e-7)
    artanh = 0.5 * jnp.log((1.0 + t) / (1.0 - t))
    y = (x / norm * artanh).astype(jnp.bfloat16)
    bn = y.shape[0]
    o_ref[...] = y.reshape(bn, D // 32, 2, 16).transpose(0, 1, 3, 2).reshape(bn, D)


def _logmap0(x):
    bn = 1000
    return pl.pallas_call(
        _logmap0_body,
        grid=(N // bn,),
        in_specs=[pl.BlockSpec((bn, D), lambda i: (i, 0))],
        out_specs=pl.BlockSpec((bn, D), lambda i: (i, 0)),
        out_shape=jax.ShapeDtypeStruct((N, D), jnp.bfloat16),
    )(x)


def _combine_body(p0_ref, p1_ref, h0_ref, a_ref, t_ref, w_ref, o_ref):
    a = a_ref[0, 0]
    th = t_ref[0, 0]
    hi = p0_ref[...] + p1_ref[...]
    support = (1.0 - a) * hi + a * h0_ref[...]
    out = th * jnp.dot(support, w_ref[...],
                       preferred_element_type=jnp.float32) + (1.0 - th) * support
    un = jnp.maximum(jnp.sqrt(jnp.sum(out * out, axis=1, keepdims=True)), 1e-15)
    e = jnp.tanh(un) * out / un
    en = jnp.maximum(jnp.sqrt(jnp.sum(e * e, axis=1, keepdims=True)), 1e-15)
    maxnorm = 1.0 - 1e-5
    o_ref[...] = jnp.where(en > maxnorm, e / en * maxnorm, e)


def _combine(p0, p1, h0, alpha, theta, weight):
    bn = 1000
    return pl.pallas_call(
        _combine_body,
        grid=(N // bn,),
        in_specs=[
            pl.BlockSpec((bn, D), lambda i: (i, 0)),
            pl.BlockSpec((bn, D), lambda i: (i, 0)),
            pl.BlockSpec((bn, D), lambda i: (i, 0)),
            pl.BlockSpec((1, 1), lambda i: (0, 0)),
            pl.BlockSpec((1, 1), lambda i: (0, 0)),
            pl.BlockSpec((D, D), lambda i: (0, 0)),
        ],
        out_specs=pl.BlockSpec((bn, D), lambda i: (i, 0)),
        out_shape=jax.ShapeDtypeStruct((N, D), jnp.float32),
    )(p0, p1, h0, alpha, theta, weight)


def _sc_body(xt_hbm, src_hbm, dst_hbm, w_hbm, out_hbm,
             src_v, dst_v, w_v, gb0, gb1, fb0, fb1, hi_sh,
             g0, g1, t0, t1, isem, dsem, wsem):
    c = lax.axis_index("c")
    s = lax.axis_index("s")
    wid = c * NS + s
    gbufs = (gb0, gb1)
    gsem = (g0, g1)
    fbufs = (fb0, fb1)
    tsem = (t0, t1)

    pltpu.async_copy(src_hbm.at[wid, 0], src_v.at[0], isem)
    pltpu.async_copy(dst_hbm.at[wid, 0], dst_v.at[0], dsem)
    pltpu.async_copy(w_hbm.at[wid, 0], w_v.at[0], wsem)

    z = jnp.zeros((16,), jnp.float32)

    def _zero_row(i, carry):
        for j in range(D // 16):
            fb0[i, pl.ds(j * 16, 16)] = z
        return carry

    lax.fori_loop(0, HCH, _zero_row, 0)

    r0 = s * ROWS_MAIN
    nzc = jnp.where(s == NS - 1, ROWS_LAST // WCH, ROWS_MAIN // WCH)

    def _zero_stripe(t, carry):
        pltpu.sync_copy(fb0.at[pl.ds(0, WCH)], hi_sh.at[pl.ds(r0 + t * WCH, WCH)])
        return carry

    lax.fori_loop(0, nzc, _zero_stripe, 0)

    @pl.when(s == NS - 1)
    def _zero_tail():
        pltpu.sync_copy(fb0.at[pl.ds(0, ROWS_LAST % WCH)],
                        hi_sh.at[pl.ds(r0 + (ROWS_LAST // WCH) * WCH,
                                       ROWS_LAST % WCH)])

    plsc.subcore_barrier()

    def _scale_half(pp, row, h, gb, fb):
        def body(ib, inner):
            base = h * HCH + ib * 16
            wv = w_v[pp, row, pl.ds(base, 16)]
            for r in range(16):
                rr = base + r
                fr = ib * 16 + r
                wi = jnp.full((16,), wv[r], dtype=jnp.float32)
                for j in range(D // 32):
                    xi = gb[rr, pl.ds(j * 16, 16)]
                    lo = jax.lax.bitcast_convert_type(
                        xi << jnp.uint32(16), jnp.float32)
                    hi2 = jax.lax.bitcast_convert_type(
                        xi & jnp.uint32(0xFFFF0000), jnp.float32)
                    fb[fr, pl.ds(j * 32, 16)] = lo * wi
                    fb[fr, pl.ds(j * 32 + 16, 16)] = hi2 * wi
            return inner

        lax.fori_loop(0, HCH // 16, body, 0)

    pltpu.make_async_copy(src_hbm.at[wid, 0], src_v.at[0], isem).wait()
    pltpu.make_async_copy(dst_hbm.at[wid, 0], dst_v.at[0], dsem).wait()
    pltpu.make_async_copy(w_hbm.at[wid, 0], w_v.at[0], wsem).wait()
    pltpu.async_copy(src_hbm.at[wid, 1], src_v.at[1], isem)
    pltpu.async_copy(dst_hbm.at[wid, 1], dst_v.at[1], dsem)
    pltpu.async_copy(w_hbm.at[wid, 1], w_v.at[1], wsem)
    pltpu.async_copy(xt_hbm.at[src_v.at[0, 0]], gbufs[0], gsem[0])

    def _step(k2, carry):
        for b in range(2):
            kk = k2 * 2 + b
            pp = (kk // BS) % 2
            row = kk % BS

            pltpu.make_async_copy(
                xt_hbm.at[src_v.at[pp, row]], gbufs[b], gsem[b]).wait()

            if b == 1:
                @pl.when(jnp.logical_and(row == BS - 1, kk < CPW - 1))
                def _():
                    q = (kk + 1) // BS
                    qq = q % 2
                    pltpu.make_async_copy(src_hbm.at[wid, q], src_v.at[qq], isem).wait()
                    pltpu.make_async_copy(dst_hbm.at[wid, q], dst_v.at[qq], dsem).wait()
                    pltpu.make_async_copy(w_hbm.at[wid, q], w_v.at[qq], wsem).wait()

            @pl.when(kk < CPW - 1)
            def _():
                kn = kk + 1
                ppn = (kn // BS) % 2
                pltpu.async_copy(
                    xt_hbm.at[src_v.at[ppn, kn % BS]], gbufs[1 - b], gsem[1 - b])

            def _drain(h):
                pltpu.make_async_copy(
                    fbufs[h], hi_sh.at[dst_v.at[pp, 2 * row + h]],
                    tsem[h]).wait()

            if b == 0:
                @pl.when(k2 > 0)
                def _():
                    _drain(0)
            else:
                _drain(0)
            _scale_half(pp, row, 0, gbufs[b], fbufs[0])
            pltpu.async_copy(fbufs[0], hi_sh.at[dst_v.at[pp, 2 * row]],
                             tsem[0], add=True)

            if b == 0:
                @pl.when(k2 > 0)
                def _():
                    _drain(1)

                @pl.when(jnp.logical_and(row == 0, jnp.logical_and(
                    kk >= BS, kk // BS + 1 < NB)))
                def _():
                    q2 = kk // BS + 1
                    qq2 = q2 % 2
                    pltpu.async_copy(src_hbm.at[wid, q2], src_v.at[qq2], isem)
                    pltpu.async_copy(dst_hbm.at[wid, q2], dst_v.at[qq2], dsem)
                    pltpu.async_copy(w_hbm.at[wid, q2], w_v.at[qq2], wsem)
            else:
                _drain(1)
            _scale_half(pp, row, 1, gbufs[b], fbufs[1])
            pltpu.async_copy(fbufs[1], hi_sh.at[dst_v.at[pp, 2 * row + 1]],
                             tsem[1], add=True)
        return carry

    lax.fori_loop(0, CPW // 2, _step, 0)
    pltpu.make_async_copy(fbufs[0], hi_sh.at[dst_v.at[1, 0]], tsem[0]).wait()
    pltpu.make_async_copy(fbufs[1], hi_sh.at[dst_v.at[1, 0]], tsem[1]).wait()

    plsc.subcore_barrier()

    def _writeout(t, carry):
        ro = r0 + t * WCH
        pltpu.sync_copy(hi_sh.at[pl.ds(ro, WCH)], fb0.at[pl.ds(0, WCH)])
        pltpu.sync_copy(fb0.at[pl.ds(0, WCH)], out_hbm.at[c].at[pl.ds(ro, WCH)])
        return carry

    lax.fori_loop(0, nzc, _writeout, 0)

    @pl.when(s == NS - 1)
    def _write_tail():
        tail = ROWS_LAST % WCH
        ro = r0 + (ROWS_LAST // WCH) * WCH
        pltpu.sync_copy(hi_sh.at[pl.ds(ro, tail)], fb0.at[pl.ds(0, tail)])
        pltpu.sync_copy(fb0.at[pl.ds(0, tail)], out_hbm.at[c].at[pl.ds(ro, tail)])


_sc_spmm = functools.partial(
    pl.kernel,
    out_type=jax.ShapeDtypeStruct((NC, N, D), jnp.float32),
    mesh=plsc.VectorSubcoreMesh(core_axis_name="c", subcore_axis_name="s"),
    compiler_params=pltpu.CompilerParams(use_tc_tiling_on_sc=False),
    scratch_types=[
        pltpu.VMEM((2, BS, CH), jnp.int32),
        pltpu.VMEM((2, 2 * BS, HCH), jnp.int32),
        pltpu.VMEM((2, BS, CH), jnp.float32),
        pltpu.VMEM((CH, D // 2), jnp.uint32),
        pltpu.VMEM((CH, D // 2), jnp.uint32),
        pltpu.VMEM((HCH, D), jnp.float32),
        pltpu.VMEM((HCH, D), jnp.float32),
        pltpu.VMEM_SHARED((N, D), jnp.float32),
        pltpu.SemaphoreType.DMA,
        pltpu.SemaphoreType.DMA,
        pltpu.SemaphoreType.DMA,
        pltpu.SemaphoreType.DMA,
        pltpu.SemaphoreType.DMA,
        pltpu.SemaphoreType.DMA,
        pltpu.SemaphoreType.DMA,
    ],
)(_sc_body)


def kernel(x, edge_index, edge_weight, h0, alpha, theta, weight):
    npad = EPAD - E
    pad_idx = jnp.arange(npad, dtype=jnp.int32) % N
    src = jnp.concatenate(
        [edge_index[0].astype(jnp.int32), pad_idx]
    ).reshape(NW, NB, BS, CH)
    dst = jnp.concatenate(
        [edge_index[1].astype(jnp.int32), pad_idx]
    ).reshape(NW, NB, 2 * BS, HCH)
    w2 = jnp.concatenate(
        [edge_weight, jnp.zeros((npad,), jnp.float32)]
    ).reshape(NW, NB, BS, CH)
    xt = _logmap0(x)
    xt_u32 = jax.lax.bitcast_convert_type(
        xt.reshape(N, D // 2, 2), jnp.uint32)
    partials = _sc_spmm(xt_u32, src, dst, w2)
    return _combine(partials[0], partials[1], h0,
                    alpha.reshape(1, 1), theta.reshape(1, 1), weight)


# speedup vs baseline: 2.3783x; 2.3783x over previous
import functools

import jax
import jax.numpy as jnp
from jax import lax
from jax.experimental import pallas as pl
from jax.experimental.pallas import tpu as pltpu
from jax.experimental.pallas import tpu_sc as plsc

N = 10000
E = 320000
D = 128

NC = 2
NS = 16
NW = NC * NS

CH = 128
CPW = 80
BS = 16
NB = CPW // BS
EPAD = NW * CPW * CH
ROWS_MAIN = 624
ROWS_LAST = 640
WCH = 104


def _logmap0_body(x_ref, o_ref):
    x = x_ref[...]
    n2 = jnp.sum(x * x, axis=1, keepdims=True)
    norm = jnp.maximum(jnp.sqrt(n2), 1e-15)
    t = jnp.clip(norm, -1.0 + 1e-7, 1.0 - 1e-7)
    artanh = 0.5 * jnp.log((1.0 + t) / (1.0 - t))
    o_ref[...] = x / norm * artanh


def _logmap0(x):
    bn = 1000
    return pl.pallas_call(
        _logmap0_body,
        grid=(N // bn,),
        in_specs=[pl.BlockSpec((bn, D), lambda i: (i, 0))],
        out_specs=pl.BlockSpec((bn, D), lambda i: (i, 0)),
        out_shape=jax.ShapeDtypeStruct((N, D), jnp.float32),
    )(x)


def _combine_body(p0_ref, p1_ref, h0_ref, a_ref, t_ref, w_ref, o_ref):
    a = a_ref[0, 0]
    th = t_ref[0, 0]
    hi = p0_ref[0] + p1_ref[0]
    support = (1.0 - a) * hi + a * h0_ref[...]
    out = th * jnp.dot(support, w_ref[...],
                       preferred_element_type=jnp.float32) + (1.0 - th) * support
    un = jnp.maximum(jnp.sqrt(jnp.sum(out * out, axis=1, keepdims=True)), 1e-15)
    e = jnp.tanh(un) * out / un
    en = jnp.maximum(jnp.sqrt(jnp.sum(e * e, axis=1, keepdims=True)), 1e-15)
    maxnorm = 1.0 - 1e-5
    o_ref[...] = jnp.where(en > maxnorm, e / en * maxnorm, e)


def _combine(partials, h0, alpha, theta, weight):
    bn = 1000
    return pl.pallas_call(
        _combine_body,
        grid=(N // bn,),
        in_specs=[
            pl.BlockSpec((1, bn, D), lambda i: (0, i, 0)),
            pl.BlockSpec((1, bn, D), lambda i: (1, i, 0)),
            pl.BlockSpec((bn, D), lambda i: (i, 0)),
            pl.BlockSpec((1, 1), lambda i: (0, 0)),
            pl.BlockSpec((1, 1), lambda i: (0, 0)),
            pl.BlockSpec((D, D), lambda i: (0, 0)),
        ],
        out_specs=pl.BlockSpec((bn, D), lambda i: (i, 0)),
        out_shape=jax.ShapeDtypeStruct((N, D), jnp.float32),
    )(partials, partials, h0, alpha, theta, weight)


def _sc_body(xt_hbm, src_hbm, dst_hbm, w_hbm, out_hbm,
             src_v, dst_v, w_v, rowbuf, rb1, hi_sh,
             g0, g1, s0, s1, isem, dsem, wsem):
    c = lax.axis_index("c")
    s = lax.axis_index("s")
    wid = c * NS + s
    bufs = (rowbuf, rb1)
    gsem = (g0, g1)
    ssem = (s0, s1)

    pltpu.async_copy(src_hbm.at[wid, 0], src_v.at[0], isem)
    pltpu.async_copy(dst_hbm.at[wid, 0], dst_v.at[0], dsem)
    pltpu.async_copy(w_hbm.at[wid, 0], w_v.at[0], wsem)

    z = jnp.zeros((16,), jnp.float32)

    def _zero_row(i, carry):
        for j in range(D // 16):
            rowbuf[i, pl.ds(j * 16, 16)] = z
        return carry

    lax.fori_loop(0, CH, _zero_row, 0)

    r0 = s * ROWS_MAIN
    nzc = jnp.where(s == NS - 1, ROWS_LAST // WCH, ROWS_MAIN // WCH)

    def _zero_stripe(t, carry):
        pltpu.sync_copy(rowbuf.at[pl.ds(0, WCH)], hi_sh.at[pl.ds(r0 + t * WCH, WCH)])
        return carry

    lax.fori_loop(0, nzc, _zero_stripe, 0)

    @pl.when(s == NS - 1)
    def _zero_tail():
        pltpu.sync_copy(rowbuf.at[pl.ds(0, ROWS_LAST % WCH)],
                        hi_sh.at[pl.ds(r0 + (ROWS_LAST // WCH) * WCH,
                                       ROWS_LAST % WCH)])

    plsc.subcore_barrier()

    def _scale(pp, row, rb):
        def body(ib, inner):
            wv = w_v[pp, row, pl.ds(ib * 16, 16)]
            for r in range(16):
                rr = ib * 16 + r
                wi = jnp.full((16,), wv[r], dtype=jnp.float32)
                for j in range(D // 16):
                    sl = pl.ds(j * 16, 16)
                    rb[rr, sl] = rb[rr, sl] * wi
            return inner

        lax.fori_loop(0, CH // 16, body, 0, unroll=2)

    pltpu.make_async_copy(src_hbm.at[wid, 0], src_v.at[0], isem).wait()
    pltpu.make_async_copy(dst_hbm.at[wid, 0], dst_v.at[0], dsem).wait()
    pltpu.make_async_copy(w_hbm.at[wid, 0], w_v.at[0], wsem).wait()
    pltpu.async_copy(src_hbm.at[wid, 1], src_v.at[1], isem)
    pltpu.async_copy(dst_hbm.at[wid, 1], dst_v.at[1], dsem)
    pltpu.async_copy(w_hbm.at[wid, 1], w_v.at[1], wsem)
    pltpu.async_copy(xt_hbm.at[src_v.at[0, 0]], bufs[0], gsem[0])

    def _step(k2, carry):
        for b in range(2):
            kk = k2 * 2 + b
            pp = (kk // BS) % 2
            row = kk % BS
            ob = 1 - b

            pltpu.make_async_copy(
                xt_hbm.at[src_v.at[pp, row]], bufs[b], gsem[b]).wait()

            if b == 0:
                @pl.when(k2 > 0)
                def _():
                    pltpu.make_async_copy(
                        bufs[ob], hi_sh.at[dst_v.at[pp, row]], ssem[ob]).wait()

                @pl.when(jnp.logical_and(row == 0, jnp.logical_and(
                    kk >= BS, kk // BS + 1 < NB)))
                def _():
                    q2 = kk // BS + 1
                    qq2 = q2 % 2
                    pltpu.async_copy(src_hbm.at[wid, q2], src_v.at[qq2], isem)
                    pltpu.async_copy(dst_hbm.at[wid, q2], dst_v.at[qq2], dsem)
                    pltpu.async_copy(w_hbm.at[wid, q2], w_v.at[qq2], wsem)
            else:
                pltpu.make_async_copy(
                    bufs[ob], hi_sh.at[dst_v.at[pp, row]], ssem[ob]).wait()

            if b == 1:
                @pl.when(jnp.logical_and(row == BS - 1, kk < CPW - 1))
                def _():
                    q = (kk + 1) // BS
                    qq = q % 2
                    pltpu.make_async_copy(src_hbm.at[wid, q], src_v.at[qq], isem).wait()
                    pltpu.make_async_copy(dst_hbm.at[wid, q], dst_v.at[qq], dsem).wait()
                    pltpu.make_async_copy(w_hbm.at[wid, q], w_v.at[qq], wsem).wait()

            @pl.when(kk < CPW - 1)
            def _():
                kn = kk + 1
                ppn = (kn // BS) % 2
                pltpu.async_copy(
                    xt_hbm.at[src_v.at[ppn, kn % BS]], bufs[ob], gsem[ob])

            _scale(pp, row, bufs[b])
            pltpu.async_copy(bufs[b], hi_sh.at[dst_v.at[pp, row]], ssem[b],
                             add=True)
        return carry

    lax.fori_loop(0, CPW // 2, _step, 0)
    pltpu.make_async_copy(bufs[1], hi_sh.at[dst_v.at[1, BS - 1]], ssem[1]).wait()

    plsc.subcore_barrier()

    def _writeout(t, carry):
        ro = r0 + t * WCH
        pltpu.sync_copy(hi_sh.at[pl.ds(ro, WCH)], rowbuf.at[pl.ds(0, WCH)])
        pltpu.sync_copy(rowbuf.at[pl.ds(0, WCH)], out_hbm.at[c].at[pl.ds(ro, WCH)])
        return carry

    lax.fori_loop(0, nzc, _writeout, 0)

    @pl.when(s == NS - 1)
    def _write_tail():
        tail = ROWS_LAST % WCH
        ro = r0 + (ROWS_LAST // WCH) * WCH
        pltpu.sync_copy(hi_sh.at[pl.ds(ro, tail)], rowbuf.at[pl.ds(0, tail)])
        pltpu.sync_copy(rowbuf.at[pl.ds(0, tail)], out_hbm.at[c].at[pl.ds(ro, tail)])


_sc_spmm = functools.partial(
    pl.kernel,
    out_type=jax.ShapeDtypeStruct((NC, N, D), jnp.float32),
    mesh=plsc.VectorSubcoreMesh(core_axis_name="c", subcore_axis_name="s"),
    scratch_types=[
        pltpu.VMEM((2, BS, CH), jnp.int32),
        pltpu.VMEM((2, BS, CH), jnp.int32),
        pltpu.VMEM((2, BS, CH), jnp.float32),
        pltpu.VMEM((CH, D), jnp.float32),
        pltpu.VMEM((CH, D), jnp.float32),
        pltpu.VMEM_SHARED((N, D), jnp.float32),
        pltpu.SemaphoreType.DMA,
        pltpu.SemaphoreType.DMA,
        pltpu.SemaphoreType.DMA,
        pltpu.SemaphoreType.DMA,
        pltpu.SemaphoreType.DMA,
        pltpu.SemaphoreType.DMA,
        pltpu.SemaphoreType.DMA,
    ],
)(_sc_body)


def kernel(x, edge_index, edge_weight, h0, alpha, theta, weight):
    npad = EPAD - E
    pad_idx = jnp.arange(npad, dtype=jnp.int32) % N
    src = jnp.concatenate(
        [edge_index[0].astype(jnp.int32), pad_idx]
    ).reshape(NW, NB, BS, CH)
    dst = jnp.concatenate(
        [edge_index[1].astype(jnp.int32), pad_idx]
    ).reshape(NW, NB, BS, CH)
    w2 = jnp.concatenate(
        [edge_weight, jnp.zeros((npad,), jnp.float32)]
    ).reshape(NW, NB, BS, CH)
    xt = _logmap0(x)
    partials = _sc_spmm(xt, src, dst, w2)
    return _combine(partials, h0,
                    alpha.reshape(1, 1), theta.reshape(1, 1), weight)


# speedup vs baseline: 2.4650x; 1.0365x over previous
import functools

import jax
import jax.numpy as jnp
from jax import lax
from jax.experimental import pallas as pl
from jax.experimental.pallas import tpu as pltpu
from jax.experimental.pallas import tpu_sc as plsc

N = 10000
E = 320000
D = 128

NC = 2
NS = 16
NW = NC * NS

CH = 128
CPW = 80
BS = 16
NB = CPW // BS
EPAD = NW * CPW * CH
ROWS_MAIN = 624
ROWS_LAST = 640
WCH = 104


def _logmap0_body(x_ref, o_ref):
    x = x_ref[...]
    n2 = jnp.sum(x * x, axis=1, keepdims=True)
    norm = jnp.maximum(jnp.sqrt(n2), 1e-15)
    t = jnp.clip(norm, -1.0 + 1e-7, 1.0 - 1e-7)
    artanh = 0.5 * jnp.log((1.0 + t) / (1.0 - t))
    o_ref[...] = x / norm * artanh


def _logmap0(x):
    bn = 2000
    return pl.pallas_call(
        _logmap0_body,
        grid=(N // bn,),
        in_specs=[pl.BlockSpec((bn, D), lambda i: (i, 0))],
        out_specs=pl.BlockSpec((bn, D), lambda i: (i, 0)),
        out_shape=jax.ShapeDtypeStruct((N, D), jnp.float32),
    )(x)


def _combine_body(p0_ref, p1_ref, h0_ref, a_ref, t_ref, w_ref, o_ref):
    a = a_ref[0, 0]
    th = t_ref[0, 0]
    hi = p0_ref[0] + p1_ref[0]
    support = (1.0 - a) * hi + a * h0_ref[...]
    out = th * jnp.dot(support, w_ref[...],
                       preferred_element_type=jnp.float32) + (1.0 - th) * support
    un = jnp.maximum(jnp.sqrt(jnp.sum(out * out, axis=1, keepdims=True)), 1e-15)
    e = jnp.tanh(un) * out / un
    en = jnp.maximum(jnp.sqrt(jnp.sum(e * e, axis=1, keepdims=True)), 1e-15)
    maxnorm = 1.0 - 1e-5
    o_ref[...] = jnp.where(en > maxnorm, e / en * maxnorm, e)


def _combine(partials, h0, alpha, theta, weight):
    bn = 2000
    return pl.pallas_call(
        _combine_body,
        grid=(N // bn,),
        in_specs=[
            pl.BlockSpec((1, bn, D), lambda i: (0, i, 0)),
            pl.BlockSpec((1, bn, D), lambda i: (1, i, 0)),
            pl.BlockSpec((bn, D), lambda i: (i, 0)),
            pl.BlockSpec((1, 1), lambda i: (0, 0)),
            pl.BlockSpec((1, 1), lambda i: (0, 0)),
            pl.BlockSpec((D, D), lambda i: (0, 0)),
        ],
        out_specs=pl.BlockSpec((bn, D), lambda i: (i, 0)),
        out_shape=jax.ShapeDtypeStruct((N, D), jnp.float32),
    )(partials, partials, h0, alpha, theta, weight)


def _sc_body(xt_hbm, src_hbm, dst_hbm, w_hbm, out_hbm,
             src_v, dst_v, w_v, rowbuf, rb1, hi_sh,
             g0, g1, s0, s1, isem, dsem, wsem):
    c = lax.axis_index("c")
    s = lax.axis_index("s")
    wid = c * NS + s
    bufs = (rowbuf, rb1)
    gsem = (g0, g1)
    ssem = (s0, s1)

    pltpu.async_copy(src_hbm.at[wid, 0], src_v.at[0], isem)
    pltpu.async_copy(dst_hbm.at[wid, 0], dst_v.at[0], dsem)
    pltpu.async_copy(w_hbm.at[wid, 0], w_v.at[0], wsem)

    z = jnp.zeros((16,), jnp.float32)

    def _zero_row(i, carry):
        for j in range(D // 16):
            rowbuf[i, pl.ds(j * 16, 16)] = z
        return carry

    lax.fori_loop(0, CH, _zero_row, 0)

    r0 = s * ROWS_MAIN
    nzc = jnp.where(s == NS - 1, ROWS_LAST // WCH, ROWS_MAIN // WCH)

    def _zero_stripe(t, carry):
        pltpu.async_copy(rowbuf.at[pl.ds(0, WCH)],
                         hi_sh.at[pl.ds(r0 + t * WCH, WCH)], g1)
        return carry

    lax.fori_loop(0, nzc, _zero_stripe, 0)

    @pl.when(s == NS - 1)
    def _zero_tail():
        pltpu.async_copy(rowbuf.at[pl.ds(0, ROWS_LAST % WCH)],
                         hi_sh.at[pl.ds(r0 + (ROWS_LAST // WCH) * WCH,
                                        ROWS_LAST % WCH)], g1)

    def _zero_drain(t, carry):
        pltpu.make_async_copy(rowbuf.at[pl.ds(0, WCH)],
                              hi_sh.at[pl.ds(r0, WCH)], g1).wait()
        return carry

    lax.fori_loop(0, nzc, _zero_drain, 0)

    @pl.when(s == NS - 1)
    def _zero_tail_drain():
        pltpu.make_async_copy(rowbuf.at[pl.ds(0, ROWS_LAST % WCH)],
                              hi_sh.at[pl.ds(r0, ROWS_LAST % WCH)], g1).wait()

    plsc.subcore_barrier()

    def _scale(pp, row, rb):
        def body(ib, inner):
            wv = w_v[pp, row, pl.ds(ib * 16, 16)]
            for r in range(16):
                rr = ib * 16 + r
                wi = jnp.full((16,), wv[r], dtype=jnp.float32)
                for j in range(D // 16):
                    sl = pl.ds(j * 16, 16)
                    rb[rr, sl] = rb[rr, sl] * wi
            return inner

        lax.fori_loop(0, CH // 16, body, 0)

    pltpu.make_async_copy(src_hbm.at[wid, 0], src_v.at[0], isem).wait()
    pltpu.make_async_copy(dst_hbm.at[wid, 0], dst_v.at[0], dsem).wait()
    pltpu.make_async_copy(w_hbm.at[wid, 0], w_v.at[0], wsem).wait()
    pltpu.async_copy(src_hbm.at[wid, 1], src_v.at[1], isem)
    pltpu.async_copy(dst_hbm.at[wid, 1], dst_v.at[1], dsem)
    pltpu.async_copy(w_hbm.at[wid, 1], w_v.at[1], wsem)
    pltpu.async_copy(xt_hbm.at[src_v.at[0, 0]], bufs[0], gsem[0])

    def _step(k2, carry):
        for b in range(2):
            kk = k2 * 2 + b
            pp = (kk // BS) % 2
            row = kk % BS
            ob = 1 - b

            pltpu.make_async_copy(
                xt_hbm.at[src_v.at[pp, row]], bufs[b], gsem[b]).wait()

            if b == 0:
                @pl.when(k2 > 0)
                def _():
                    pltpu.make_async_copy(
                        bufs[ob], hi_sh.at[dst_v.at[pp, row]], ssem[ob]).wait()

                @pl.when(jnp.logical_and(row == 0, jnp.logical_and(
                    kk >= BS, kk // BS + 1 < NB)))
                def _():
                    q2 = kk // BS + 1
                    qq2 = q2 % 2
                    pltpu.async_copy(src_hbm.at[wid, q2], src_v.at[qq2], isem)
                    pltpu.async_copy(dst_hbm.at[wid, q2], dst_v.at[qq2], dsem)
                    pltpu.async_copy(w_hbm.at[wid, q2], w_v.at[qq2], wsem)
            else:
                pltpu.make_async_copy(
                    bufs[ob], hi_sh.at[dst_v.at[pp, row]], ssem[ob]).wait()

            if b == 1:
                @pl.when(jnp.logical_and(row == BS - 1, kk < CPW - 1))
                def _():
                    q = (kk + 1) // BS
                    qq = q % 2
                    pltpu.make_async_copy(src_hbm.at[wid, q], src_v.at[qq], isem).wait()
                    pltpu.make_async_copy(dst_hbm.at[wid, q], dst_v.at[qq], dsem).wait()
                    pltpu.make_async_copy(w_hbm.at[wid, q], w_v.at[qq], wsem).wait()

            @pl.when(kk < CPW - 1)
            def _():
                kn = kk + 1
                ppn = (kn // BS) % 2
                pltpu.async_copy(
                    xt_hbm.at[src_v.at[ppn, kn % BS]], bufs[ob], gsem[ob])

            _scale(pp, row, bufs[b])
            pltpu.async_copy(bufs[b], hi_sh.at[dst_v.at[pp, row]], ssem[b],
                             add=True)
        return carry

    lax.fori_loop(0, CPW // 2, _step, 0)
    pltpu.make_async_copy(bufs[1], hi_sh.at[dst_v.at[1, BS - 1]], ssem[1]).wait()

    plsc.subcore_barrier()

    def _writeout(t, carry):
        ro = r0 + t * WCH
        pltpu.sync_copy(hi_sh.at[pl.ds(ro, WCH)], rowbuf.at[pl.ds(0, WCH)])
        pltpu.sync_copy(rowbuf.at[pl.ds(0, WCH)], out_hbm.at[c].at[pl.ds(ro, WCH)])
        return carry

    lax.fori_loop(0, nzc, _writeout, 0)

    @pl.when(s == NS - 1)
    def _write_tail():
        tail = ROWS_LAST % WCH
        ro = r0 + (ROWS_LAST // WCH) * WCH
        pltpu.sync_copy(hi_sh.at[pl.ds(ro, tail)], rowbuf.at[pl.ds(0, tail)])
        pltpu.sync_copy(rowbuf.at[pl.ds(0, tail)], out_hbm.at[c].at[pl.ds(ro, tail)])


_sc_spmm = functools.partial(
    pl.kernel,
    out_type=jax.ShapeDtypeStruct((NC, N, D), jnp.float32),
    mesh=plsc.VectorSubcoreMesh(core_axis_name="c", subcore_axis_name="s"),
    scratch_types=[
        pltpu.VMEM((2, BS, CH), jnp.int32),
        pltpu.VMEM((2, BS, CH), jnp.int32),
        pltpu.VMEM((2, BS, CH), jnp.float32),
        pltpu.VMEM((CH, D), jnp.float32),
        pltpu.VMEM((CH, D), jnp.float32),
        pltpu.VMEM_SHARED((N, D), jnp.float32),
        pltpu.SemaphoreType.DMA,
        pltpu.SemaphoreType.DMA,
        pltpu.SemaphoreType.DMA,
        pltpu.SemaphoreType.DMA,
        pltpu.SemaphoreType.DMA,
        pltpu.SemaphoreType.DMA,
        pltpu.SemaphoreType.DMA,
    ],
)(_sc_body)


def kernel(x, edge_index, edge_weight, h0, alpha, theta, weight):
    npad = EPAD - E
    pad_idx = jnp.arange(npad, dtype=jnp.int32) % N
    src = jnp.concatenate(
        [edge_index[0].astype(jnp.int32), pad_idx]
    ).reshape(NW, NB, BS, CH)
    dst = jnp.concatenate(
        [edge_index[1].astype(jnp.int32), pad_idx]
    ).reshape(NW, NB, BS, CH)
    w2 = jnp.concatenate(
        [edge_weight, jnp.zeros((npad,), jnp.float32)]
    ).reshape(NW, NB, BS, CH)
    xt = _logmap0(x)
    partials = _sc_spmm(xt, src, dst, w2)
    return _combine(partials, h0,
                    alpha.reshape(1, 1), theta.reshape(1, 1), weight)
